# Initial kernel scaffold; baseline (speedup 1.0000x reference)
import jax
import jax.numpy as jnp
from jax.experimental import pallas as pl


def kernel(x, edge_index, W_in, b_in, plasticity, syn, in_proj_w, in_proj_b, out_proj_w, out_proj_b, W_out, b_out):
    raise NotImplementedError("write your pallas kernel here")


# speedup vs baseline: 7.0219x; 7.0219x over previous
import functools

import jax
import jax.numpy as jnp
from jax import lax
from jax.experimental import pallas as pl
from jax.experimental.pallas import tpu as pltpu
from jax.experimental.pallas import tpu_sc as plsc

_NC = 2
_NS = 16


def _mm_bias_block(x_ref, w_ref, b_ref, o_ref):
    o_ref[...] = (
        jnp.dot(x_ref[...], w_ref[...], preferred_element_type=jnp.float32)
        + b_ref[...]
    )


def _merge_block(s_ref, a_ref, b_ref, o_ref):
    t = a_ref[...] + b_ref[...]
    t = jnp.where(t >= 0.0, t, 0.01 * t)
    o_ref[...] = t * s_ref[0, 0]


def _merge_mm_block(a_ref, b_ref, m_ref, c_ref, o_ref):
    t = a_ref[...] + b_ref[...]
    t = jnp.where(t >= 0.0, t, 0.01 * t)
    o_ref[...] = (
        jnp.dot(t, m_ref[...], preferred_element_type=jnp.float32)
        + c_ref[...]
    )


def _in_transform(x, w_t, b):
    n, d = x.shape
    h = w_t.shape[1]
    bn = 2000
    return pl.pallas_call(
        _mm_bias_block,
        grid=(n // bn,),
        in_specs=[
            pl.BlockSpec((bn, d), lambda i: (i, 0)),
            pl.BlockSpec((d, h), lambda i: (0, 0)),
            pl.BlockSpec((1, h), lambda i: (0, 0)),
        ],
        out_specs=pl.BlockSpec((bn, h), lambda i: (i, 0)),
        out_shape=jax.ShapeDtypeStruct((n, h), jnp.float32),
    )(x, w_t, b)


def _merge(s, p0, p1):
    n, h = p0.shape
    bn = 2000
    return pl.pallas_call(
        _merge_block,
        grid=(n // bn,),
        in_specs=[
            pl.BlockSpec(memory_space=pltpu.SMEM),
            pl.BlockSpec((bn, h), lambda i: (i, 0)),
            pl.BlockSpec((bn, h), lambda i: (i, 0)),
        ],
        out_specs=pl.BlockSpec((bn, h), lambda i: (i, 0)),
        out_shape=jax.ShapeDtypeStruct((n, h), jnp.float32),
    )(s, p0, p1)


def _final(p0, p1, m_t, c):
    n, h = p0.shape
    o = m_t.shape[1]
    bn = 2000
    return pl.pallas_call(
        _merge_mm_block,
        grid=(n // bn,),
        in_specs=[
            pl.BlockSpec((bn, h), lambda i: (i, 0)),
            pl.BlockSpec((bn, h), lambda i: (i, 0)),
            pl.BlockSpec((h, o), lambda i: (0, 0)),
            pl.BlockSpec((1, o), lambda i: (0, 0)),
        ],
        out_specs=pl.BlockSpec((bn, o), lambda i: (i, 0)),
        out_shape=jax.ShapeDtypeStruct((n, o), jnp.float32),
    )(p0, p1, m_t, c)


def _sc_propagate(table, src3, dst3, zeros_nh):
    n, h = table.shape
    ch, k = src3.shape[1], src3.shape[2]
    rpt = (-(-n // _NS) + 7) // 8 * 8
    rpt_last = n - (_NS - 1) * rpt
    assert rpt_last > 0 and rpt_last % 8 == 0

    mesh = plsc.VectorSubcoreMesh(core_axis_name="c", subcore_axis_name="s")

    @functools.partial(
        pl.kernel,
        mesh=mesh,
        out_type=[
            jax.ShapeDtypeStruct((n, h), jnp.float32),
            jax.ShapeDtypeStruct((n, h), jnp.float32),
        ],
        scratch_types=[
            pltpu.VMEM((ch, k), jnp.int32),
            pltpu.VMEM((ch, k), jnp.int32),
            pltpu.VMEM((k, h), jnp.float32),
            pltpu.VMEM_SHARED((n, h), jnp.float32),
        ],
    )
    def run(table_hbm, src_hbm, dst_hbm, z_hbm, p0_hbm, p1_hbm,
            src_v, dst_v, rows_v, acc_sh):
        cid = lax.axis_index("c")
        sid = lax.axis_index("s")
        wid = cid * _NS + sid

        @pl.when(sid < _NS - 1)
        def _():
            pltpu.sync_copy(z_hbm.at[pl.ds(sid * rpt, rpt)],
                            acc_sh.at[pl.ds(sid * rpt, rpt)])

        @pl.when(sid == _NS - 1)
        def _():
            pltpu.sync_copy(z_hbm.at[pl.ds((_NS - 1) * rpt, rpt_last)],
                            acc_sh.at[pl.ds((_NS - 1) * rpt, rpt_last)])

        pltpu.sync_copy(src_hbm.at[wid], src_v)
        pltpu.sync_copy(dst_hbm.at[wid], dst_v)
        plsc.subcore_barrier()

        def body(j, carry):
            pltpu.sync_copy(table_hbm.at[src_v.at[j]], rows_v)
            pltpu.sync_copy(rows_v, acc_sh.at[dst_v.at[j]], add=True)
            return carry

        lax.fori_loop(0, ch, body, 0)
        plsc.subcore_barrier()

        for core, out_hbm in ((0, p0_hbm), (1, p1_hbm)):
            @pl.when(jnp.logical_and(cid == core, sid < _NS - 1))
            def _(out_hbm=out_hbm):
                pltpu.sync_copy(acc_sh.at[pl.ds(sid * rpt, rpt)],
                                out_hbm.at[pl.ds(sid * rpt, rpt)])

            @pl.when(jnp.logical_and(cid == core, sid == _NS - 1))
            def _(out_hbm=out_hbm):
                pltpu.sync_copy(acc_sh.at[pl.ds((_NS - 1) * rpt, rpt_last)],
                                out_hbm.at[pl.ds((_NS - 1) * rpt, rpt_last)])

    return run(table, src3, dst3, zeros_nh)


def kernel(x, edge_index, W_in, b_in, plasticity, syn, in_proj_w, in_proj_b,
           out_proj_w, out_proj_b, W_out, b_out):
    n, d = x.shape
    h = W_in.shape[0]
    e = edge_index.shape[1]
    nw = _NC * _NS
    ept = e // nw
    k = 8
    for cand in range(8, 129, 8):
        if ept % cand == 0:
            k = cand
    ch = ept // k

    sig = jax.nn.sigmoid
    gate = sig(plasticity) * sig(syn)

    w_in_t = (W_in * gate[0]).T
    b0 = (b_in * gate[0]).reshape(1, h)

    w_v = in_proj_w[2 * h:]
    b_v = in_proj_b[2 * h:]
    m_t = (W_out @ out_proj_w @ w_v).T
    c = ((b_v @ out_proj_w.T + out_proj_b) @ W_out.T + b_out).reshape(1, -1)

    ei = edge_index.astype(jnp.int32)
    src3 = ei[0].reshape(nw, ch, k)
    dst3 = ei[1].reshape(nw, ch, k)
    z = jnp.zeros((n, h), jnp.float32)

    table = _in_transform(x, w_in_t, b0)
    p0, p1 = _sc_propagate(table, src3, dst3, z)
    table = _merge(gate[1].reshape(1, 1), p0, p1)
    p0, p1 = _sc_propagate(table, src3, dst3, z)
    table = _merge(gate[2].reshape(1, 1), p0, p1)
    p0, p1 = _sc_propagate(table, src3, dst3, z)
    return _final(p0, p1, m_t, c)
